# Initial kernel scaffold; baseline (speedup 1.0000x reference)
import jax
import jax.numpy as jnp
from jax.experimental import pallas as pl


def kernel(x, edge_index, boundary_index, interior_index, W_b1, b_b1, W_b2, b_b2, W_i1, b_i1, W_i2, b_i2, W_c1, b_c1, W_c2, b_c2, W_c3, b_c3):
    raise NotImplementedError("write your pallas kernel here")


# speedup vs baseline: 11.5837x; 11.5837x over previous
import functools

import jax
import jax.numpy as jnp
from jax import lax
from jax.experimental import pallas as pl
from jax.experimental.pallas import tpu as pltpu
from jax.experimental.pallas import tpu_sc as plsc

NN = 100000
EE = 1600000
NC = 2
NS = 16
NPAD = 100352
RPT = NPAD // NS
CHUNK = 128
JC = EE // CHUNK
F32 = jnp.float32

@functools.cache
def _mesh():
    return plsc.VectorSubcoreMesh(core_axis_name="c", subcore_axis_name="s",
                                  num_cores=NC, num_subcores=NS)


def _span(s, total):
    lo = total // NS
    extra = total - lo * NS
    t0 = s * lo + jnp.minimum(s, extra)
    cnt = lo + jnp.where(s < extra, 1, 0)
    return t0, cnt


@functools.cache
def _flags_deg_k():
    return pl.kernel(
        _flags_deg_body,
        out_type=(
            jax.ShapeDtypeStruct((NPAD, 1), F32),
            jax.ShapeDtypeStruct((NPAD, 1), F32),
            jax.ShapeDtypeStruct((NC, NPAD, 1), F32),
        ),
        mesh=_mesh(),
        compiler_params=pltpu.CompilerParams(use_tc_tiling_on_sc=False),
        scratch_types=(
            pltpu.VMEM_SHARED((NPAD, 1), F32),
            pltpu.VMEM_SHARED((NPAD, 1), F32),
            pltpu.VMEM_SHARED((NPAD, 1), F32),
            pltpu.VMEM((128, 1), F32),
            pltpu.VMEM((80, 1), F32),
            pltpu.VMEM((120, 1), F32),
            pltpu.VMEM((128,), jnp.int32),
            pltpu.VMEM((80,), jnp.int32),
            pltpu.VMEM((120,), jnp.int32),
        ),
    )


def _flags_deg_body(dst, bidx, iidx, z1, ones, wb_out, wi_out, dg_out,
               wb_acc, wi_acc, dg_acc, one128, one80, one120,
               ib128, ib80, ib120):
    c = lax.axis_index("c")
    s = lax.axis_index("s")
    base = s * RPT
    pltpu.sync_copy(z1.at[pl.ds(base, RPT), :], dg_acc.at[pl.ds(base, RPT), :])
    pltpu.sync_copy(ones.at[pl.ds(0, 128), :], one128)
    pltpu.sync_copy(ones.at[pl.ds(0, 80), :], one80)
    pltpu.sync_copy(ones.at[pl.ds(0, 120), :], one120)

    @pl.when(c == 0)
    def _zero_flags():
        pltpu.sync_copy(z1.at[pl.ds(base, RPT), :], wb_acc.at[pl.ds(base, RPT), :])
        pltpu.sync_copy(z1.at[pl.ds(base, RPT), :], wi_acc.at[pl.ds(base, RPT), :])

    plsc.subcore_barrier()

    t0, cnt = _span(s, JC // NC)

    def deg_step(i, carry):
        j = c * (JC // NC) + t0 + i
        pltpu.sync_copy(dst.at[pl.ds(j * CHUNK, CHUNK)], ib128)
        pltpu.sync_copy(one128, dg_acc.at[ib128], add=True)
        return carry

    lax.fori_loop(0, cnt, deg_step, 0)

    @pl.when(c == 0)
    def _flags():
        b0, bcnt = _span(s, 125)

        def b_step(i, carry):
            pltpu.sync_copy(bidx.at[pl.ds((b0 + i) * 80, 80)], ib80)
            pltpu.sync_copy(one80, wb_acc.at[ib80], add=True)
            return carry

        lax.fori_loop(0, bcnt, b_step, 0)
        i0, icnt = _span(s, 750)

        def i_step(i, carry):
            pltpu.sync_copy(iidx.at[pl.ds((i0 + i) * 120, 120)], ib120)
            pltpu.sync_copy(one120, wi_acc.at[ib120], add=True)
            return carry

        lax.fori_loop(0, icnt, i_step, 0)

    plsc.subcore_barrier()
    pltpu.sync_copy(dg_acc.at[pl.ds(base, RPT), :], dg_out.at[c, pl.ds(base, RPT), :])

    @pl.when(c == 0)
    def _out_flags():
        pltpu.sync_copy(wb_acc.at[pl.ds(base, RPT), :], wb_out.at[pl.ds(base, RPT), :])
        pltpu.sync_copy(wi_acc.at[pl.ds(base, RPT), :], wi_out.at[pl.ds(base, RPT), :])


@functools.cache
def _spmm32_k():
    return pl.kernel(
        _spmm32_body,
        out_type=jax.ShapeDtypeStruct((NC, NPAD, 16), F32),
        mesh=_mesh(),
        compiler_params=pltpu.CompilerParams(use_tc_tiling_on_sc=False),
        scratch_types=(
            pltpu.VMEM_SHARED((NPAD, 16), F32),
            pltpu.VMEM((CHUNK,), jnp.int32),
            pltpu.VMEM((CHUNK,), jnp.int32),
            pltpu.VMEM((CHUNK, 16), F32),
            pltpu.SemaphoreType.DMA,
        ),
    )


def _spmm32_body(src, dst, y2, z16, out, acc, sbuf, dbuf, rows, gsem):
    c = lax.axis_index("c")
    s = lax.axis_index("s")
    base = s * RPT
    pltpu.sync_copy(z16.at[pl.ds(base, RPT), :], acc.at[pl.ds(base, RPT), :])
    plsc.subcore_barrier()

    t0, cnt = _span(s, JC)
    off = c * NN

    def step(i, carry):
        j = t0 + i
        pltpu.sync_copy(src.at[pl.ds(j * CHUNK, CHUNK)], sbuf)
        pltpu.sync_copy(dst.at[pl.ds(j * CHUNK, CHUNK)], dbuf)
        for k in range(CHUNK // 16):
            sbuf[pl.ds(k * 16, 16)] = sbuf[pl.ds(k * 16, 16)] + off
        pltpu.async_copy(y2.at[sbuf], rows, gsem).wait()
        pltpu.sync_copy(rows, acc.at[dbuf], add=True)
        return carry

    lax.fori_loop(0, cnt, step, 0)
    plsc.subcore_barrier()
    pltpu.sync_copy(acc.at[pl.ds(base, RPT), :], out.at[c, pl.ds(base, RPT), :])


@functools.cache
def _spmm1_k():
    return pl.kernel(
        _spmm1_body,
        out_type=jax.ShapeDtypeStruct((NC, NPAD, 1), F32),
        mesh=_mesh(),
        compiler_params=pltpu.CompilerParams(use_tc_tiling_on_sc=False),
        scratch_types=(
            pltpu.VMEM_SHARED((NPAD, 1), F32),
            pltpu.VMEM((CHUNK,), jnp.int32),
            pltpu.VMEM((CHUNK,), jnp.int32),
            pltpu.VMEM((CHUNK, 1), F32),
            pltpu.SemaphoreType.DMA,
        ),
    )


def _spmm1_body(src, dst, y3, z1, out, acc, sbuf, dbuf, rows, gsem):
    c = lax.axis_index("c")
    s = lax.axis_index("s")
    base = s * RPT
    pltpu.sync_copy(z1.at[pl.ds(base, RPT), :], acc.at[pl.ds(base, RPT), :])
    plsc.subcore_barrier()

    t0, cnt = _span(s, JC // NC)

    def step(i, carry):
        j = c * (JC // NC) + t0 + i
        pltpu.sync_copy(src.at[pl.ds(j * CHUNK, CHUNK)], sbuf)
        pltpu.sync_copy(dst.at[pl.ds(j * CHUNK, CHUNK)], dbuf)
        pltpu.async_copy(y3.at[sbuf], rows, gsem).wait()
        pltpu.sync_copy(rows, acc.at[dbuf], add=True)
        return carry

    lax.fori_loop(0, cnt, step, 0)
    plsc.subcore_barrier()
    pltpu.sync_copy(acc.at[pl.ds(base, RPT), :], out.at[c, pl.ds(base, RPT), :])


BM = 2000
_GRID = NN // BM


def _w(shape):
    return pl.BlockSpec(shape, lambda i: (0,) * len(shape))


def _stage2_body(x_ref, wb_ref, wi_ref, dg_ref,
                 wb1, bb1, wb2, bb2, wi1, bi1, wi2, bi2, wc1,
                 y_ref, dinv_ref):
    xb = x_ref[...]
    mb = jnp.maximum(
        jnp.dot(xb, wb1[...], preferred_element_type=F32) + bb1[...], 0.0)
    mb = jnp.dot(mb, wb2[...], preferred_element_type=F32) + bb2[...]
    mi = jnp.maximum(
        jnp.dot(xb[:, :2], wi1[...], preferred_element_type=F32) + bi1[...], 0.0)
    mi = jnp.dot(mi, wi2[...], preferred_element_type=F32) + bi2[...]
    h = jnp.where(wi_ref[...] > 0.5, mi, jnp.where(wb_ref[...] > 0.5, mb, 0.0))
    h = jnp.maximum(h, 0.0)
    deg = dg_ref[0] + dg_ref[1] + 1.0
    dinv = lax.rsqrt(deg)
    y = jnp.dot(h, wc1[...], preferred_element_type=F32) * dinv
    y_ref[0] = y[:, :16]
    y_ref[1] = y[:, 16:]
    dinv_ref[...] = dinv


def _stage2(x, wb, wi, dgp, W_b1, b_b1, W_b2, b_b2, W_i1, b_i1, W_i2, b_i2, W_c1):
    return pl.pallas_call(
        _stage2_body,
        grid=(_GRID,),
        in_specs=[
            pl.BlockSpec((BM, 3), lambda i: (i, 0)),
            pl.BlockSpec((BM, 1), lambda i: (i, 0)),
            pl.BlockSpec((BM, 1), lambda i: (i, 0)),
            pl.BlockSpec((NC, BM, 1), lambda i: (0, i, 0)),
            _w((3, 16)), _w((1, 16)), _w((16, 16)), _w((1, 16)),
            _w((2, 16)), _w((1, 16)), _w((16, 16)), _w((1, 16)),
            _w((16, 32)),
        ],
        out_specs=[
            pl.BlockSpec((NC, BM, 16), lambda i: (0, i, 0)),
            pl.BlockSpec((BM, 1), lambda i: (i, 0)),
        ],
        out_shape=[
            jax.ShapeDtypeStruct((NC, NN, 16), F32),
            jax.ShapeDtypeStruct((NN, 1), F32),
        ],
    )(x, wb, wi, dgp, W_b1, b_b1, W_b2, b_b2, W_i1, b_i1, W_i2, b_i2, W_c1)


def _mid_body(acc_ref, y_ref, dinv_ref, w_ref, b_ref, o_ref):
    a = jnp.concatenate([acc_ref[0], acc_ref[1]], axis=1)
    yv = jnp.concatenate([y_ref[0], y_ref[1]], axis=1)
    dinv = dinv_ref[...]
    h = jnp.maximum(dinv * (a + yv) + b_ref[...], 0.0)
    y = jnp.dot(h, w_ref[...], preferred_element_type=F32) * dinv
    if o_ref.shape[-1] == 16:
        o_ref[0] = y[:, :16]
        o_ref[1] = y[:, 16:]
    else:
        o_ref[...] = y


def _mid(acc, y, dinv, W, b):
    fo = W.shape[1]
    if fo == 32:
        out_spec = pl.BlockSpec((NC, BM, 16), lambda i: (0, i, 0))
        out_shape = jax.ShapeDtypeStruct((NC, NN, 16), F32)
    else:
        out_spec = pl.BlockSpec((BM, fo), lambda i: (i, 0))
        out_shape = jax.ShapeDtypeStruct((NN, fo), F32)
    return pl.pallas_call(
        _mid_body,
        grid=(_GRID,),
        in_specs=[
            pl.BlockSpec((NC, BM, 16), lambda i: (0, i, 0)),
            pl.BlockSpec((NC, BM, 16), lambda i: (0, i, 0)),
            pl.BlockSpec((BM, 1), lambda i: (i, 0)),
            _w((32, fo)), _w((1, 32)),
        ],
        out_specs=out_spec,
        out_shape=out_shape,
    )(acc, y, dinv, W, b)


def _final_body(acc_ref, y_ref, dinv_ref, b_ref, o_ref):
    a = acc_ref[0] + acc_ref[1] + y_ref[...]
    o_ref[...] = dinv_ref[...] * a + b_ref[...]


def _final(acc3, y3, dinv, b):
    return pl.pallas_call(
        _final_body,
        grid=(_GRID,),
        in_specs=[
            pl.BlockSpec((NC, BM, 1), lambda i: (0, i, 0)),
            pl.BlockSpec((BM, 1), lambda i: (i, 0)),
            pl.BlockSpec((BM, 1), lambda i: (i, 0)),
            _w((1, 1)),
        ],
        out_specs=pl.BlockSpec((BM, 1), lambda i: (i, 0)),
        out_shape=jax.ShapeDtypeStruct((NN, 1), F32),
    )(acc3, y3, dinv, b)


def kernel(x, edge_index, boundary_index, interior_index,
           W_b1, b_b1, W_b2, b_b2, W_i1, b_i1, W_i2, b_i2,
           W_c1, b_c1, W_c2, b_c2, W_c3, b_c3):
    src = edge_index[0]
    dst = edge_index[1]
    z1 = jnp.zeros((NPAD, 1), F32)
    z16 = jnp.zeros((NPAD, 16), F32)
    ones = jnp.ones((128, 1), F32)

    wb, wi, dgp = _flags_deg_k()(dst, boundary_index, interior_index, z1, ones)
    y1, dinv = _stage2(x, wb, wi, dgp,
                       W_b1, b_b1.reshape(1, 16), W_b2, b_b2.reshape(1, 16),
                       W_i1, b_i1.reshape(1, 16), W_i2, b_i2.reshape(1, 16),
                       W_c1)
    acc1 = _spmm32_k()(src, dst, y1.reshape(NC * NN, 16), z16)
    y2 = _mid(acc1, y1, dinv, W_c2, b_c1.reshape(1, 32))
    acc2 = _spmm32_k()(src, dst, y2.reshape(NC * NN, 16), z16)
    y3 = _mid(acc2, y2, dinv, W_c3, b_c2.reshape(1, 32))
    acc3 = _spmm1_k()(src, dst, y3, z1)
    out = _final(acc3, y3, dinv, b_c3.reshape(1, 1))
    return out


# speedup vs baseline: 22.6281x; 1.9535x over previous
import functools

import jax
import jax.numpy as jnp
from jax import lax
from jax.experimental import pallas as pl
from jax.experimental.pallas import tpu as pltpu
from jax.experimental.pallas import tpu_sc as plsc

NN = 100000
EE = 1600000
NC = 2
NS = 16
NPAD = 100352
RPT = NPAD // NS
CHUNK = 512
JC = EE // CHUNK
NW = NC * NS
F32 = jnp.float32

@functools.cache
def _mesh():
    return plsc.VectorSubcoreMesh(core_axis_name="c", subcore_axis_name="s",
                                  num_cores=NC, num_subcores=NS)


def _span(w, total, nw):
    lo = total // nw
    extra = total - lo * nw
    t0 = w * lo + jnp.minimum(w, extra)
    cnt = lo + jnp.where(w < extra, 1, 0)
    return t0, cnt


@functools.cache
def _flags_deg_k():
    return pl.kernel(
        _flags_deg_body,
        out_type=(
            jax.ShapeDtypeStruct((NPAD, 1), F32),
            jax.ShapeDtypeStruct((NPAD, 1), F32),
            jax.ShapeDtypeStruct((NC, NPAD, 1), F32),
        ),
        mesh=_mesh(),
        compiler_params=pltpu.CompilerParams(use_tc_tiling_on_sc=False),
        scratch_types=(
            pltpu.VMEM_SHARED((NPAD, 1), F32),
            pltpu.VMEM_SHARED((NPAD, 1), F32),
            pltpu.VMEM_SHARED((NPAD, 1), F32),
            pltpu.VMEM((CHUNK, 1), F32),
            pltpu.VMEM((400, 1), F32),
            pltpu.VMEM((CHUNK,), jnp.int32),
            pltpu.VMEM((400,), jnp.int32),
        ),
    )


def _flags_deg_body(dst, bidx, iidx, z1, ones, wb_out, wi_out, dg_out,
               wb_acc, wi_acc, dg_acc, onec, one400, ibc, ib400):
    c = lax.axis_index("c")
    s = lax.axis_index("s")
    w = c * NS + s
    base = s * RPT
    pltpu.sync_copy(z1.at[pl.ds(base, RPT), :], dg_acc.at[pl.ds(base, RPT), :])
    pltpu.sync_copy(ones.at[pl.ds(0, CHUNK), :], onec)
    pltpu.sync_copy(ones.at[pl.ds(0, 400), :], one400)

    @pl.when(c == 0)
    def _zero_flags():
        pltpu.sync_copy(z1.at[pl.ds(base, RPT), :], wb_acc.at[pl.ds(base, RPT), :])
        pltpu.sync_copy(z1.at[pl.ds(base, RPT), :], wi_acc.at[pl.ds(base, RPT), :])

    plsc.subcore_barrier()

    t0, cnt = _span(w, JC, NW)

    def deg_step(i, carry):
        j = t0 + i
        pltpu.sync_copy(dst.at[pl.ds(j * CHUNK, CHUNK)], ibc)
        pltpu.sync_copy(onec, dg_acc.at[ibc], add=True)
        return carry

    lax.fori_loop(0, cnt, deg_step, 0)

    @pl.when(c == 0)
    def _flags():
        b0, bcnt = _span(s, 25, NS)

        def b_step(i, carry):
            pltpu.sync_copy(bidx.at[pl.ds((b0 + i) * 400, 400)], ib400)
            pltpu.sync_copy(one400, wb_acc.at[ib400], add=True)
            return carry

        lax.fori_loop(0, bcnt, b_step, 0)
        i0, icnt = _span(s, 225, NS)

        def i_step(i, carry):
            pltpu.sync_copy(iidx.at[pl.ds((i0 + i) * 400, 400)], ib400)
            pltpu.sync_copy(one400, wi_acc.at[ib400], add=True)
            return carry

        lax.fori_loop(0, icnt, i_step, 0)

    plsc.subcore_barrier()
    pltpu.sync_copy(dg_acc.at[pl.ds(base, RPT), :], dg_out.at[c, pl.ds(base, RPT), :])

    @pl.when(c == 0)
    def _out_flags():
        pltpu.sync_copy(wb_acc.at[pl.ds(base, RPT), :], wb_out.at[pl.ds(base, RPT), :])
        pltpu.sync_copy(wi_acc.at[pl.ds(base, RPT), :], wi_out.at[pl.ds(base, RPT), :])


@functools.cache
def _spmm32_k():
    return pl.kernel(
        _spmm32_body,
        out_type=jax.ShapeDtypeStruct((NC, NPAD, 16), F32),
        mesh=_mesh(),
        compiler_params=pltpu.CompilerParams(use_tc_tiling_on_sc=False),
        scratch_types=(
            pltpu.VMEM_SHARED((NPAD, 16), F32),
            pltpu.VMEM((CHUNK,), jnp.int32),
            pltpu.VMEM((CHUNK,), jnp.int32),
            pltpu.VMEM((CHUNK, 16), F32),
            pltpu.SemaphoreType.DMA,
        ),
    )


def _spmm32_body(src2, dst, y2, z16, out, acc, sbuf, dbuf, rows, gsem):
    c = lax.axis_index("c")
    s = lax.axis_index("s")
    base = s * RPT
    pltpu.sync_copy(z16.at[pl.ds(base, RPT), :], acc.at[pl.ds(base, RPT), :])
    plsc.subcore_barrier()

    t0, cnt = _span(s, JC, NS)
    sbase = c * EE

    def step(i, carry):
        j = t0 + i
        pltpu.sync_copy(src2.at[pl.ds(sbase + j * CHUNK, CHUNK)], sbuf)
        pltpu.sync_copy(dst.at[pl.ds(j * CHUNK, CHUNK)], dbuf)
        pltpu.async_copy(y2.at[sbuf], rows, gsem).wait()
        pltpu.sync_copy(rows, acc.at[dbuf], add=True)
        return carry

    lax.fori_loop(0, cnt, step, 0)
    plsc.subcore_barrier()
    pltpu.sync_copy(acc.at[pl.ds(base, RPT), :], out.at[c, pl.ds(base, RPT), :])


@functools.cache
def _spmm1_k():
    return pl.kernel(
        _spmm1_body,
        out_type=jax.ShapeDtypeStruct((NC, NPAD, 1), F32),
        mesh=_mesh(),
        compiler_params=pltpu.CompilerParams(use_tc_tiling_on_sc=False),
        scratch_types=(
            pltpu.VMEM_SHARED((NPAD, 1), F32),
            pltpu.VMEM((CHUNK,), jnp.int32),
            pltpu.VMEM((CHUNK,), jnp.int32),
            pltpu.VMEM((CHUNK, 1), F32),
            pltpu.SemaphoreType.DMA,
        ),
    )


def _spmm1_body(src, dst, y3, z1, out, acc, sbuf, dbuf, rows, gsem):
    c = lax.axis_index("c")
    s = lax.axis_index("s")
    base = s * RPT
    pltpu.sync_copy(z1.at[pl.ds(base, RPT), :], acc.at[pl.ds(base, RPT), :])
    plsc.subcore_barrier()

    t0, cnt = _span(c * NS + s, JC, NW)

    def step(i, carry):
        j = t0 + i
        pltpu.sync_copy(src.at[pl.ds(j * CHUNK, CHUNK)], sbuf)
        pltpu.sync_copy(dst.at[pl.ds(j * CHUNK, CHUNK)], dbuf)
        pltpu.async_copy(y3.at[sbuf], rows, gsem).wait()
        pltpu.sync_copy(rows, acc.at[dbuf], add=True)
        return carry

    lax.fori_loop(0, cnt, step, 0)
    plsc.subcore_barrier()
    pltpu.sync_copy(acc.at[pl.ds(base, RPT), :], out.at[c, pl.ds(base, RPT), :])


BM = 2000
_GRID = NN // BM


def _w(shape):
    return pl.BlockSpec(shape, lambda i: (0,) * len(shape))


def _stage2_body(x_ref, wb_ref, wi_ref, dg_ref,
                 wb1, bb1, wb2, bb2, wi1, bi1, wi2, bi2, wc1,
                 y_ref, dinv_ref):
    xb = x_ref[...]
    mb = jnp.maximum(
        jnp.dot(xb, wb1[...], preferred_element_type=F32) + bb1[...], 0.0)
    mb = jnp.dot(mb, wb2[...], preferred_element_type=F32) + bb2[...]
    mi = jnp.maximum(
        jnp.dot(xb[:, :2], wi1[...], preferred_element_type=F32) + bi1[...], 0.0)
    mi = jnp.dot(mi, wi2[...], preferred_element_type=F32) + bi2[...]
    h = jnp.where(wi_ref[...] > 0.5, mi, jnp.where(wb_ref[...] > 0.5, mb, 0.0))
    h = jnp.maximum(h, 0.0)
    deg = dg_ref[0] + dg_ref[1] + 1.0
    dinv = lax.rsqrt(deg)
    y = jnp.dot(h, wc1[...], preferred_element_type=F32) * dinv
    y_ref[0] = y[:, :16]
    y_ref[1] = y[:, 16:]
    dinv_ref[...] = dinv


def _stage2(x, wb, wi, dgp, W_b1, b_b1, W_b2, b_b2, W_i1, b_i1, W_i2, b_i2, W_c1):
    return pl.pallas_call(
        _stage2_body,
        grid=(_GRID,),
        in_specs=[
            pl.BlockSpec((BM, 3), lambda i: (i, 0)),
            pl.BlockSpec((BM, 1), lambda i: (i, 0)),
            pl.BlockSpec((BM, 1), lambda i: (i, 0)),
            pl.BlockSpec((NC, BM, 1), lambda i: (0, i, 0)),
            _w((3, 16)), _w((1, 16)), _w((16, 16)), _w((1, 16)),
            _w((2, 16)), _w((1, 16)), _w((16, 16)), _w((1, 16)),
            _w((16, 32)),
        ],
        out_specs=[
            pl.BlockSpec((NC, BM, 16), lambda i: (0, i, 0)),
            pl.BlockSpec((BM, 1), lambda i: (i, 0)),
        ],
        out_shape=[
            jax.ShapeDtypeStruct((NC, NN, 16), F32),
            jax.ShapeDtypeStruct((NN, 1), F32),
        ],
    )(x, wb, wi, dgp, W_b1, b_b1, W_b2, b_b2, W_i1, b_i1, W_i2, b_i2, W_c1)


def _mid_body(acc_ref, y_ref, dinv_ref, w_ref, b_ref, o_ref):
    a = jnp.concatenate([acc_ref[0], acc_ref[1]], axis=1)
    yv = jnp.concatenate([y_ref[0], y_ref[1]], axis=1)
    dinv = dinv_ref[...]
    h = jnp.maximum(dinv * (a + yv) + b_ref[...], 0.0)
    y = jnp.dot(h, w_ref[...], preferred_element_type=F32) * dinv
    if o_ref.shape[-1] == 16:
        o_ref[0] = y[:, :16]
        o_ref[1] = y[:, 16:]
    else:
        o_ref[...] = y


def _mid(acc, y, dinv, W, b):
    fo = W.shape[1]
    if fo == 32:
        out_spec = pl.BlockSpec((NC, BM, 16), lambda i: (0, i, 0))
        out_shape = jax.ShapeDtypeStruct((NC, NN, 16), F32)
    else:
        out_spec = pl.BlockSpec((BM, fo), lambda i: (i, 0))
        out_shape = jax.ShapeDtypeStruct((NN, fo), F32)
    return pl.pallas_call(
        _mid_body,
        grid=(_GRID,),
        in_specs=[
            pl.BlockSpec((NC, BM, 16), lambda i: (0, i, 0)),
            pl.BlockSpec((NC, BM, 16), lambda i: (0, i, 0)),
            pl.BlockSpec((BM, 1), lambda i: (i, 0)),
            _w((32, fo)), _w((1, 32)),
        ],
        out_specs=out_spec,
        out_shape=out_shape,
    )(acc, y, dinv, W, b)


def _final_body(acc_ref, y_ref, dinv_ref, b_ref, o_ref):
    a = acc_ref[0] + acc_ref[1] + y_ref[...]
    o_ref[...] = dinv_ref[...] * a + b_ref[...]


def _final(acc3, y3, dinv, b):
    return pl.pallas_call(
        _final_body,
        grid=(_GRID,),
        in_specs=[
            pl.BlockSpec((NC, BM, 1), lambda i: (0, i, 0)),
            pl.BlockSpec((BM, 1), lambda i: (i, 0)),
            pl.BlockSpec((BM, 1), lambda i: (i, 0)),
            _w((1, 1)),
        ],
        out_specs=pl.BlockSpec((BM, 1), lambda i: (i, 0)),
        out_shape=jax.ShapeDtypeStruct((NN, 1), F32),
    )(acc3, y3, dinv, b)


def kernel(x, edge_index, boundary_index, interior_index,
           W_b1, b_b1, W_b2, b_b2, W_i1, b_i1, W_i2, b_i2,
           W_c1, b_c1, W_c2, b_c2, W_c3, b_c3):
    src = edge_index[0]
    dst = edge_index[1]
    z1 = jnp.zeros((NPAD, 1), F32)
    z16 = jnp.zeros((NPAD, 16), F32)
    ones = jnp.ones((CHUNK, 1), F32)

    src2 = jnp.concatenate([src, src + NN])
    wb, wi, dgp = _flags_deg_k()(dst, boundary_index, interior_index, z1, ones)
    y1, dinv = _stage2(x, wb, wi, dgp,
                       W_b1, b_b1.reshape(1, 16), W_b2, b_b2.reshape(1, 16),
                       W_i1, b_i1.reshape(1, 16), W_i2, b_i2.reshape(1, 16),
                       W_c1)
    acc1 = _spmm32_k()(src2, dst, y1.reshape(NC * NN, 16), z16)
    y2 = _mid(acc1, y1, dinv, W_c2, b_c1.reshape(1, 32))
    acc2 = _spmm32_k()(src2, dst, y2.reshape(NC * NN, 16), z16)
    y3 = _mid(acc2, y2, dinv, W_c3, b_c2.reshape(1, 32))
    acc3 = _spmm1_k()(src, dst, y3, z1)
    out = _final(acc3, y3, dinv, b_c3.reshape(1, 1))
    return out


# speedup vs baseline: 22.8735x; 1.0108x over previous
import functools

import jax
import jax.numpy as jnp
from jax import lax
from jax.experimental import pallas as pl
from jax.experimental.pallas import tpu as pltpu
from jax.experimental.pallas import tpu_sc as plsc

NN = 100000
EE = 1600000
NC = 2
NS = 16
NPAD = 100352
RPT = NPAD // NS
CHUNK = 512
SUP = 4
NW = NC * NS
JCP = 3200
EPAD = JCP * CHUNK
SPT32 = JCP // (NS * SUP)
SPT1 = JCP // (NW * SUP)
F32 = jnp.float32

@functools.cache
def _mesh():
    return plsc.VectorSubcoreMesh(core_axis_name="c", subcore_axis_name="s",
                                  num_cores=NC, num_subcores=NS)


def _span(w, total, nw):
    lo = total // nw
    extra = total - lo * nw
    t0 = w * lo + jnp.minimum(w, extra)
    cnt = lo + jnp.where(w < extra, 1, 0)
    return t0, cnt


@functools.cache
def _flags_deg_k():
    return pl.kernel(
        _flags_deg_body,
        out_type=(
            jax.ShapeDtypeStruct((NPAD, 1), F32),
            jax.ShapeDtypeStruct((NPAD, 1), F32),
            jax.ShapeDtypeStruct((NC, NPAD, 1), F32),
        ),
        mesh=_mesh(),
        compiler_params=pltpu.CompilerParams(use_tc_tiling_on_sc=False),
        scratch_types=(
            pltpu.VMEM_SHARED((NPAD, 1), F32),
            pltpu.VMEM_SHARED((NPAD, 1), F32),
            pltpu.VMEM_SHARED((NPAD, 1), F32),
            pltpu.VMEM((CHUNK, 1), F32),
            pltpu.VMEM((400, 1), F32),
            pltpu.VMEM((CHUNK,), jnp.int32),
            pltpu.VMEM((400,), jnp.int32),
        ),
    )


def _flags_deg_body(dst, bidx, iidx, z1, ones, wb_out, wi_out, dg_out,
               wb_acc, wi_acc, dg_acc, onec, one400, ibc, ib400):
    c = lax.axis_index("c")
    s = lax.axis_index("s")
    w = c * NS + s
    base = s * RPT
    pltpu.sync_copy(z1.at[pl.ds(base, RPT), :], dg_acc.at[pl.ds(base, RPT), :])
    pltpu.sync_copy(ones.at[pl.ds(0, CHUNK), :], onec)
    pltpu.sync_copy(ones.at[pl.ds(0, 400), :], one400)

    @pl.when(c == 0)
    def _zero_flags():
        pltpu.sync_copy(z1.at[pl.ds(base, RPT), :], wb_acc.at[pl.ds(base, RPT), :])
        pltpu.sync_copy(z1.at[pl.ds(base, RPT), :], wi_acc.at[pl.ds(base, RPT), :])

    plsc.subcore_barrier()

    t0 = w * (SPT1 * SUP)

    def deg_step(i, carry):
        j = t0 + i
        pltpu.sync_copy(dst.at[pl.ds(j * CHUNK, CHUNK)], ibc)
        pltpu.sync_copy(onec, dg_acc.at[ibc], add=True)
        return carry

    lax.fori_loop(0, SPT1 * SUP, deg_step, 0)

    @pl.when(c == 0)
    def _flags():
        b0, bcnt = _span(s, 25, NS)

        def b_step(i, carry):
            pltpu.sync_copy(bidx.at[pl.ds((b0 + i) * 400, 400)], ib400)
            pltpu.sync_copy(one400, wb_acc.at[ib400], add=True)
            return carry

        lax.fori_loop(0, bcnt, b_step, 0)
        i0, icnt = _span(s, 225, NS)

        def i_step(i, carry):
            pltpu.sync_copy(iidx.at[pl.ds((i0 + i) * 400, 400)], ib400)
            pltpu.sync_copy(one400, wi_acc.at[ib400], add=True)
            return carry

        lax.fori_loop(0, icnt, i_step, 0)

    plsc.subcore_barrier()
    pltpu.sync_copy(dg_acc.at[pl.ds(base, RPT), :], dg_out.at[c, pl.ds(base, RPT), :])

    @pl.when(c == 0)
    def _out_flags():
        pltpu.sync_copy(wb_acc.at[pl.ds(base, RPT), :], wb_out.at[pl.ds(base, RPT), :])
        pltpu.sync_copy(wi_acc.at[pl.ds(base, RPT), :], wi_out.at[pl.ds(base, RPT), :])


@functools.cache
def _spmm32_k():
    return pl.kernel(
        _spmm32_body,
        out_type=jax.ShapeDtypeStruct((NC, NPAD, 16), F32),
        mesh=_mesh(),
        compiler_params=pltpu.CompilerParams(use_tc_tiling_on_sc=False),
        scratch_types=(
            pltpu.VMEM_SHARED((NPAD, 16), F32),
            pltpu.VMEM((SUP, CHUNK), jnp.int32),
            pltpu.VMEM((SUP, CHUNK), jnp.int32),
            pltpu.VMEM((CHUNK, 16), F32),
            pltpu.VMEM((CHUNK, 16), F32),
            pltpu.SemaphoreType.DMA,
            pltpu.SemaphoreType.DMA,
        ),
    )


def _spmm32_body(srcR, dstR, y2, z16, out, acc, sidx, didx, rows0, rows1,
                 g0, g1):
    c = lax.axis_index("c")
    s = lax.axis_index("s")
    base = s * RPT
    pltpu.sync_copy(z16.at[pl.ds(base, RPT), :], acc.at[pl.ds(base, RPT), :])
    plsc.subcore_barrier()

    t0 = s * (SPT32 * SUP)
    sjc = c * JCP
    rows = (rows0, rows1)
    gs = (g0, g1)

    def outer(r, carry):
        j = t0 + r * SUP
        pltpu.sync_copy(srcR.at[pl.ds(sjc + j, SUP), :], sidx)
        pltpu.sync_copy(dstR.at[pl.ds(j, SUP), :], didx)
        cps = [None] * SUP
        cps[0] = pltpu.async_copy(y2.at[sidx.at[0]], rows0, g0)
        for k in range(SUP):
            if k + 1 < SUP:
                cps[k + 1] = pltpu.async_copy(
                    y2.at[sidx.at[k + 1]], rows[(k + 1) % 2], gs[(k + 1) % 2])
            cps[k].wait()
            pltpu.sync_copy(rows[k % 2], acc.at[didx.at[k]], add=True)
        return carry

    lax.fori_loop(0, SPT32, outer, 0)
    plsc.subcore_barrier()
    pltpu.sync_copy(acc.at[pl.ds(base, RPT), :], out.at[c, pl.ds(base, RPT), :])


@functools.cache
def _spmm1_k():
    return pl.kernel(
        _spmm1_body,
        out_type=jax.ShapeDtypeStruct((NC, NPAD, 1), F32),
        mesh=_mesh(),
        compiler_params=pltpu.CompilerParams(use_tc_tiling_on_sc=False),
        scratch_types=(
            pltpu.VMEM_SHARED((NPAD, 1), F32),
            pltpu.VMEM((SUP, CHUNK), jnp.int32),
            pltpu.VMEM((SUP, CHUNK), jnp.int32),
            pltpu.VMEM((CHUNK, 1), F32),
            pltpu.VMEM((CHUNK, 1), F32),
            pltpu.SemaphoreType.DMA,
            pltpu.SemaphoreType.DMA,
        ),
    )


def _spmm1_body(srcR, dstR, y3, z1, out, acc, sidx, didx, rows0, rows1,
                g0, g1):
    c = lax.axis_index("c")
    s = lax.axis_index("s")
    base = s * RPT
    pltpu.sync_copy(z1.at[pl.ds(base, RPT), :], acc.at[pl.ds(base, RPT), :])
    plsc.subcore_barrier()

    t0 = (c * NS + s) * (SPT1 * SUP)
    rows = (rows0, rows1)
    gs = (g0, g1)

    def outer(r, carry):
        j = t0 + r * SUP
        pltpu.sync_copy(srcR.at[pl.ds(j, SUP), :], sidx)
        pltpu.sync_copy(dstR.at[pl.ds(j, SUP), :], didx)
        cps = [None] * SUP
        cps[0] = pltpu.async_copy(y3.at[sidx.at[0]], rows0, g0)
        for k in range(SUP):
            if k + 1 < SUP:
                cps[k + 1] = pltpu.async_copy(
                    y3.at[sidx.at[k + 1]], rows[(k + 1) % 2], gs[(k + 1) % 2])
            cps[k].wait()
            pltpu.sync_copy(rows[k % 2], acc.at[didx.at[k]], add=True)
        return carry

    lax.fori_loop(0, SPT1, outer, 0)
    plsc.subcore_barrier()
    pltpu.sync_copy(acc.at[pl.ds(base, RPT), :], out.at[c, pl.ds(base, RPT), :])


BM = 2000
_GRID = NN // BM


def _w(shape):
    return pl.BlockSpec(shape, lambda i: (0,) * len(shape))


def _stage2_body(x_ref, wb_ref, wi_ref, dg_ref,
                 wb1, bb1, wb2, bb2, wi1, bi1, wi2, bi2, wc1,
                 y_ref, dinv_ref):
    xb = x_ref[...]
    mb = jnp.maximum(
        jnp.dot(xb, wb1[...], preferred_element_type=F32) + bb1[...], 0.0)
    mb = jnp.dot(mb, wb2[...], preferred_element_type=F32) + bb2[...]
    mi = jnp.maximum(
        jnp.dot(xb[:, :2], wi1[...], preferred_element_type=F32) + bi1[...], 0.0)
    mi = jnp.dot(mi, wi2[...], preferred_element_type=F32) + bi2[...]
    h = jnp.where(wi_ref[...] > 0.5, mi, jnp.where(wb_ref[...] > 0.5, mb, 0.0))
    h = jnp.maximum(h, 0.0)
    deg = dg_ref[0] + dg_ref[1] + 1.0
    dinv = lax.rsqrt(deg)
    y = jnp.dot(h, wc1[...], preferred_element_type=F32) * dinv
    y_ref[0] = y[:, :16]
    y_ref[1] = y[:, 16:]
    dinv_ref[...] = dinv


def _stage2(x, wb, wi, dgp, W_b1, b_b1, W_b2, b_b2, W_i1, b_i1, W_i2, b_i2, W_c1):
    return pl.pallas_call(
        _stage2_body,
        grid=(_GRID,),
        in_specs=[
            pl.BlockSpec((BM, 3), lambda i: (i, 0)),
            pl.BlockSpec((BM, 1), lambda i: (i, 0)),
            pl.BlockSpec((BM, 1), lambda i: (i, 0)),
            pl.BlockSpec((NC, BM, 1), lambda i: (0, i, 0)),
            _w((3, 16)), _w((1, 16)), _w((16, 16)), _w((1, 16)),
            _w((2, 16)), _w((1, 16)), _w((16, 16)), _w((1, 16)),
            _w((16, 32)),
        ],
        out_specs=[
            pl.BlockSpec((NC, BM, 16), lambda i: (0, i, 0)),
            pl.BlockSpec((BM, 1), lambda i: (i, 0)),
        ],
        out_shape=[
            jax.ShapeDtypeStruct((NC, NN, 16), F32),
            jax.ShapeDtypeStruct((NN, 1), F32),
        ],
    )(x, wb, wi, dgp, W_b1, b_b1, W_b2, b_b2, W_i1, b_i1, W_i2, b_i2, W_c1)


def _mid_body(acc_ref, y_ref, dinv_ref, w_ref, b_ref, o_ref):
    a = jnp.concatenate([acc_ref[0], acc_ref[1]], axis=1)
    yv = jnp.concatenate([y_ref[0], y_ref[1]], axis=1)
    dinv = dinv_ref[...]
    h = jnp.maximum(dinv * (a + yv) + b_ref[...], 0.0)
    y = jnp.dot(h, w_ref[...], preferred_element_type=F32) * dinv
    if o_ref.shape[-1] == 16:
        o_ref[0] = y[:, :16]
        o_ref[1] = y[:, 16:]
    else:
        o_ref[...] = y


def _mid(acc, y, dinv, W, b):
    fo = W.shape[1]
    if fo == 32:
        out_spec = pl.BlockSpec((NC, BM, 16), lambda i: (0, i, 0))
        out_shape = jax.ShapeDtypeStruct((NC, NN, 16), F32)
    else:
        out_spec = pl.BlockSpec((BM, fo), lambda i: (i, 0))
        out_shape = jax.ShapeDtypeStruct((NN, fo), F32)
    return pl.pallas_call(
        _mid_body,
        grid=(_GRID,),
        in_specs=[
            pl.BlockSpec((NC, BM, 16), lambda i: (0, i, 0)),
            pl.BlockSpec((NC, BM, 16), lambda i: (0, i, 0)),
            pl.BlockSpec((BM, 1), lambda i: (i, 0)),
            _w((32, fo)), _w((1, 32)),
        ],
        out_specs=out_spec,
        out_shape=out_shape,
    )(acc, y, dinv, W, b)


def _final_body(acc_ref, y_ref, dinv_ref, b_ref, o_ref):
    a = acc_ref[0] + acc_ref[1] + y_ref[...]
    o_ref[...] = dinv_ref[...] * a + b_ref[...]


def _final(acc3, y3, dinv, b):
    return pl.pallas_call(
        _final_body,
        grid=(_GRID,),
        in_specs=[
            pl.BlockSpec((NC, BM, 1), lambda i: (0, i, 0)),
            pl.BlockSpec((BM, 1), lambda i: (i, 0)),
            pl.BlockSpec((BM, 1), lambda i: (i, 0)),
            _w((1, 1)),
        ],
        out_specs=pl.BlockSpec((BM, 1), lambda i: (i, 0)),
        out_shape=jax.ShapeDtypeStruct((NN, 1), F32),
    )(acc3, y3, dinv, b)


def kernel(x, edge_index, boundary_index, interior_index,
           W_b1, b_b1, W_b2, b_b2, W_i1, b_i1, W_i2, b_i2,
           W_c1, b_c1, W_c2, b_c2, W_c3, b_c3):
    src = edge_index[0]
    dst = edge_index[1]
    z1 = jnp.zeros((NPAD, 1), F32)
    z16 = jnp.zeros((NPAD, 16), F32)
    ones = jnp.ones((CHUNK, 1), F32)

    pad = EPAD - EE
    src_p = jnp.concatenate([src, jnp.zeros((pad,), jnp.int32)])
    dst_p = jnp.concatenate([dst, jnp.full((pad,), NN, jnp.int32)])
    src2R = jnp.concatenate([src_p, src_p + NN]).reshape(2 * JCP, CHUNK)
    srcR = src_p.reshape(JCP, CHUNK)
    dstR = dst_p.reshape(JCP, CHUNK)
    wb, wi, dgp = _flags_deg_k()(dst_p, boundary_index, interior_index, z1, ones)
    y1, dinv = _stage2(x, wb, wi, dgp,
                       W_b1, b_b1.reshape(1, 16), W_b2, b_b2.reshape(1, 16),
                       W_i1, b_i1.reshape(1, 16), W_i2, b_i2.reshape(1, 16),
                       W_c1)
    acc1 = _spmm32_k()(src2R, dstR, y1.reshape(NC * NN, 16), z16)
    y2 = _mid(acc1, y1, dinv, W_c2, b_c1.reshape(1, 32))
    acc2 = _spmm32_k()(src2R, dstR, y2.reshape(NC * NN, 16), z16)
    y3 = _mid(acc2, y2, dinv, W_c3, b_c2.reshape(1, 32))
    acc3 = _spmm1_k()(srcR, dstR, y3, z1)
    out = _final(acc3, y3, dinv, b_c3.reshape(1, 1))
    return out


# speedup vs baseline: 25.1082x; 1.0977x over previous
import functools

import jax
import jax.numpy as jnp
from jax import lax
from jax.experimental import pallas as pl
from jax.experimental.pallas import tpu as pltpu
from jax.experimental.pallas import tpu_sc as plsc

NN = 100000
EE = 1600000
NC = 2
NS = 16
NPAD = 100352
RPT = NPAD // NS
CHUNK = 512
SUP = 4
NW = NC * NS
JCP = 3200
EPAD = JCP * CHUNK
SPT32 = JCP // (NS * SUP)
SPT1 = JCP // (NW * SUP)
F32 = jnp.float32


@functools.cache
def _mesh():
    return plsc.VectorSubcoreMesh(core_axis_name="c", subcore_axis_name="s",
                                  num_cores=NC, num_subcores=NS)


def _span(w, total, nw):
    lo = total // nw
    extra = total - lo * nw
    t0 = w * lo + jnp.minimum(w, extra)
    cnt = lo + jnp.where(w < extra, 1, 0)
    return t0, cnt


@functools.cache
def _flags_deg_k():
    return pl.kernel(
        _flags_deg_body,
        out_type=jax.ShapeDtypeStruct((4, NPAD, 1), F32),
        mesh=_mesh(),
        compiler_params=pltpu.CompilerParams(use_tc_tiling_on_sc=False),
        scratch_types=(
            pltpu.VMEM_SHARED((NPAD, 1), F32),
            pltpu.VMEM_SHARED((NPAD, 1), F32),
            pltpu.VMEM_SHARED((NPAD, 1), F32),
            pltpu.VMEM((CHUNK, 1), F32),
            pltpu.VMEM((400, 1), F32),
            pltpu.VMEM((CHUNK,), jnp.int32),
            pltpu.VMEM((400,), jnp.int32),
        ),
    )


def _flags_deg_body(dst, bidx, iidx, z1, ones, f_out,
                    wb_acc, wi_acc, dg_acc, onec, one400, ibc, ib400):
    c = lax.axis_index("c")
    s = lax.axis_index("s")
    w = c * NS + s
    base = s * RPT
    pltpu.sync_copy(z1.at[pl.ds(base, RPT), :], dg_acc.at[pl.ds(base, RPT), :])
    pltpu.sync_copy(ones.at[pl.ds(0, CHUNK), :], onec)
    pltpu.sync_copy(ones.at[pl.ds(0, 400), :], one400)

    @pl.when(c == 0)
    def _zero_flags():
        pltpu.sync_copy(z1.at[pl.ds(base, RPT), :], wb_acc.at[pl.ds(base, RPT), :])
        pltpu.sync_copy(z1.at[pl.ds(base, RPT), :], wi_acc.at[pl.ds(base, RPT), :])

    plsc.subcore_barrier()

    t0 = w * (SPT1 * SUP)

    def deg_step(i, carry):
        j = t0 + i
        pltpu.sync_copy(dst.at[pl.ds(j * CHUNK, CHUNK)], ibc)
        pltpu.sync_copy(onec, dg_acc.at[ibc], add=True)
        return carry

    lax.fori_loop(0, SPT1 * SUP, deg_step, 0)

    @pl.when(c == 0)
    def _flags():
        b0, bcnt = _span(s, 25, NS)

        def b_step(i, carry):
            pltpu.sync_copy(bidx.at[pl.ds((b0 + i) * 400, 400)], ib400)
            pltpu.sync_copy(one400, wb_acc.at[ib400], add=True)
            return carry

        lax.fori_loop(0, bcnt, b_step, 0)
        i0, icnt = _span(s, 225, NS)

        def i_step(i, carry):
            pltpu.sync_copy(iidx.at[pl.ds((i0 + i) * 400, 400)], ib400)
            pltpu.sync_copy(one400, wi_acc.at[ib400], add=True)
            return carry

        lax.fori_loop(0, icnt, i_step, 0)

    plsc.subcore_barrier()

    @pl.when(c == 0)
    def _out_dg0():
        pltpu.sync_copy(dg_acc.at[pl.ds(base, RPT), :],
                        f_out.at[2, pl.ds(base, RPT), :])

    @pl.when(c == 1)
    def _out_dg1():
        pltpu.sync_copy(dg_acc.at[pl.ds(base, RPT), :],
                        f_out.at[3, pl.ds(base, RPT), :])

    @pl.when(c == 0)
    def _out_flags():
        pltpu.sync_copy(wb_acc.at[pl.ds(base, RPT), :],
                        f_out.at[0, pl.ds(base, RPT), :])
        pltpu.sync_copy(wi_acc.at[pl.ds(base, RPT), :],
                        f_out.at[1, pl.ds(base, RPT), :])


def _make_spmm_split(width):
    def body(srcR, dstR, table, zeros, out, acc, sidx, didx, rows0, rows1,
             g0, g1):
        c = lax.axis_index("c")
        s = lax.axis_index("s")
        base = s * RPT
        pltpu.sync_copy(zeros.at[pl.ds(base, RPT), :], acc.at[pl.ds(base, RPT), :])
        plsc.subcore_barrier()

        t0 = (c * NS + s) * (SPT1 * SUP)
        rows = (rows0, rows1)
        gs = (g0, g1)

        def outer(r, carry):
            j = t0 + r * SUP
            pltpu.sync_copy(srcR.at[pl.ds(j, SUP), :], sidx)
            pltpu.sync_copy(dstR.at[pl.ds(j, SUP), :], didx)
            cps = [None] * SUP
            cps[0] = pltpu.async_copy(table.at[sidx.at[0]], rows0, g0)
            for k in range(SUP):
                if k + 1 < SUP:
                    cps[k + 1] = pltpu.async_copy(
                        table.at[sidx.at[k + 1]], rows[(k + 1) % 2],
                        gs[(k + 1) % 2])
                cps[k].wait()
                pltpu.sync_copy(rows[k % 2], acc.at[didx.at[k]], add=True)
            return carry

        lax.fori_loop(0, SPT1, outer, 0)
        plsc.subcore_barrier()
        pltpu.sync_copy(acc.at[pl.ds(base, RPT), :],
                        out.at[c, pl.ds(base, RPT), :])

    return pl.kernel(
        body,
        out_type=jax.ShapeDtypeStruct((NC, NPAD, width), F32),
        mesh=_mesh(),
        compiler_params=pltpu.CompilerParams(use_tc_tiling_on_sc=False),
        scratch_types=(
            pltpu.VMEM_SHARED((NPAD, width), F32),
            pltpu.VMEM((SUP, CHUNK), jnp.int32),
            pltpu.VMEM((SUP, CHUNK), jnp.int32),
            pltpu.VMEM((CHUNK, width), F32),
            pltpu.VMEM((CHUNK, width), F32),
            pltpu.SemaphoreType.DMA,
            pltpu.SemaphoreType.DMA,
        ),
    )


@functools.cache
def _spmm16_k():
    return _make_spmm_split(16)


@functools.cache
def _spmm1_k():
    return _make_spmm_split(1)


@functools.cache
def _spmm32_k():
    return pl.kernel(
        _spmm32_body,
        out_type=jax.ShapeDtypeStruct((NC, NPAD, 16), F32),
        mesh=_mesh(),
        compiler_params=pltpu.CompilerParams(use_tc_tiling_on_sc=False),
        scratch_types=(
            pltpu.VMEM_SHARED((NPAD, 16), F32),
            pltpu.VMEM((SUP, CHUNK), jnp.int32),
            pltpu.VMEM((SUP, CHUNK), jnp.int32),
            pltpu.VMEM((CHUNK, 16), F32),
            pltpu.VMEM((CHUNK, 16), F32),
            pltpu.SemaphoreType.DMA,
            pltpu.SemaphoreType.DMA,
        ),
    )


def _spmm32_body(srcR, dstR, y2, z16, out, acc, sidx, didx, rows0, rows1,
                 g0, g1):
    c = lax.axis_index("c")
    s = lax.axis_index("s")
    base = s * RPT
    pltpu.sync_copy(z16.at[pl.ds(base, RPT), :], acc.at[pl.ds(base, RPT), :])
    plsc.subcore_barrier()

    t0 = s * (SPT32 * SUP)
    rows = (rows0, rows1)
    gs = (g0, g1)
    sjc = c * JCP

    def outer(r, carry):
        j = t0 + r * SUP
        pltpu.sync_copy(srcR.at[pl.ds(sjc + j, SUP), :], sidx)
        pltpu.sync_copy(dstR.at[pl.ds(j, SUP), :], didx)
        cps = [None] * SUP
        cps[0] = pltpu.async_copy(y2.at[sidx.at[0]], rows0, g0)
        for k in range(SUP):
            if k + 1 < SUP:
                cps[k + 1] = pltpu.async_copy(
                    y2.at[sidx.at[k + 1]], rows[(k + 1) % 2], gs[(k + 1) % 2])
            cps[k].wait()
            pltpu.sync_copy(rows[k % 2], acc.at[didx.at[k]], add=True)
        return carry

    lax.fori_loop(0, SPT32, outer, 0)
    plsc.subcore_barrier()
    pltpu.sync_copy(acc.at[pl.ds(base, RPT), :], out.at[c, pl.ds(base, RPT), :])


BM = 4000
_GRID = NN // BM


def _w(shape):
    return pl.BlockSpec(shape, lambda i: (0,) * len(shape))


def _stage2_body(x_ref, f_ref,
                 wb1, bb1, wb2, bb2, wi1, bi1, wi2, bi2,
                 z_ref, dinv_ref):
    xb = x_ref[...]
    mb = jnp.maximum(
        jnp.dot(xb, wb1[...], preferred_element_type=F32) + bb1[...], 0.0)
    mb = jnp.dot(mb, wb2[...], preferred_element_type=F32) + bb2[...]
    mi = jnp.maximum(
        jnp.dot(xb[:, :2], wi1[...], preferred_element_type=F32) + bi1[...], 0.0)
    mi = jnp.dot(mi, wi2[...], preferred_element_type=F32) + bi2[...]
    h = jnp.where(f_ref[1] > 0.5, mi, jnp.where(f_ref[0] > 0.5, mb, 0.0))
    h = jnp.maximum(h, 0.0)
    deg = f_ref[2] + f_ref[3] + 1.0
    dinv = lax.rsqrt(deg)
    z_ref[...] = h * dinv
    dinv_ref[...] = dinv


def _stage2(x, f4, W_b1, b_b1, W_b2, b_b2, W_i1, b_i1, W_i2, b_i2):
    return pl.pallas_call(
        _stage2_body,
        grid=(_GRID,),
        in_specs=[
            pl.BlockSpec((BM, 3), lambda i: (i, 0)),
            pl.BlockSpec((4, BM, 1), lambda i: (0, i, 0)),
            _w((3, 16)), _w((1, 16)), _w((16, 16)), _w((1, 16)),
            _w((2, 16)), _w((1, 16)), _w((16, 16)), _w((1, 16)),
        ],
        out_specs=[
            pl.BlockSpec((BM, 16), lambda i: (i, 0)),
            pl.BlockSpec((BM, 1), lambda i: (i, 0)),
        ],
        out_shape=[
            jax.ShapeDtypeStruct((NN, 16), F32),
            jax.ShapeDtypeStruct((NN, 1), F32),
        ],
    )(x, f4, W_b1, b_b1, W_b2, b_b2, W_i1, b_i1, W_i2, b_i2)


def _gcn1_body(agg_ref, z_ref, dinv_ref, w1_ref, b1_ref, w2_ref, o_ref):
    dinv = dinv_ref[...]
    u = dinv * (agg_ref[0] + agg_ref[1] + z_ref[...])
    h2 = jnp.maximum(
        jnp.dot(u, w1_ref[...], preferred_element_type=F32) + b1_ref[...], 0.0)
    y = jnp.dot(h2, w2_ref[...], preferred_element_type=F32) * dinv
    o_ref[0] = y[:, :16]
    o_ref[1] = y[:, 16:]


def _gcn1(agg1, z, dinv, W_c1, b_c1, W_c2):
    return pl.pallas_call(
        _gcn1_body,
        grid=(_GRID,),
        in_specs=[
            pl.BlockSpec((NC, BM, 16), lambda i: (0, i, 0)),
            pl.BlockSpec((BM, 16), lambda i: (i, 0)),
            pl.BlockSpec((BM, 1), lambda i: (i, 0)),
            _w((16, 32)), _w((1, 32)), _w((32, 32)),
        ],
        out_specs=pl.BlockSpec((NC, BM, 16), lambda i: (0, i, 0)),
        out_shape=jax.ShapeDtypeStruct((NC, NN, 16), F32),
    )(agg1, z, dinv, W_c1, b_c1, W_c2)


def _mid_body(acc_ref, y_ref, dinv_ref, w_ref, b_ref, o_ref):
    a = jnp.concatenate([acc_ref[0], acc_ref[1]], axis=1)
    yv = jnp.concatenate([y_ref[0], y_ref[1]], axis=1)
    dinv = dinv_ref[...]
    h = jnp.maximum(dinv * (a + yv) + b_ref[...], 0.0)
    o_ref[...] = jnp.dot(h, w_ref[...], preferred_element_type=F32) * dinv


def _mid(acc, y, dinv, W, b):
    fo = W.shape[1]
    return pl.pallas_call(
        _mid_body,
        grid=(_GRID,),
        in_specs=[
            pl.BlockSpec((NC, BM, 16), lambda i: (0, i, 0)),
            pl.BlockSpec((NC, BM, 16), lambda i: (0, i, 0)),
            pl.BlockSpec((BM, 1), lambda i: (i, 0)),
            _w((32, fo)), _w((1, 32)),
        ],
        out_specs=pl.BlockSpec((BM, fo), lambda i: (i, 0)),
        out_shape=jax.ShapeDtypeStruct((NN, fo), F32),
    )(acc, y, dinv, W, b)


def _final_body(acc_ref, y_ref, dinv_ref, b_ref, o_ref):
    a = acc_ref[0] + acc_ref[1] + y_ref[...]
    o_ref[...] = dinv_ref[...] * a + b_ref[...]


def _final(acc3, y3, dinv, b):
    return pl.pallas_call(
        _final_body,
        grid=(_GRID,),
        in_specs=[
            pl.BlockSpec((NC, BM, 1), lambda i: (0, i, 0)),
            pl.BlockSpec((BM, 1), lambda i: (i, 0)),
            pl.BlockSpec((BM, 1), lambda i: (i, 0)),
            _w((1, 1)),
        ],
        out_specs=pl.BlockSpec((BM, 1), lambda i: (i, 0)),
        out_shape=jax.ShapeDtypeStruct((NN, 1), F32),
    )(acc3, y3, dinv, b)


def kernel(x, edge_index, boundary_index, interior_index,
           W_b1, b_b1, W_b2, b_b2, W_i1, b_i1, W_i2, b_i2,
           W_c1, b_c1, W_c2, b_c2, W_c3, b_c3):
    src = edge_index[0]
    dst = edge_index[1]
    z1 = jnp.zeros((NPAD, 1), F32)
    z16 = jnp.zeros((NPAD, 16), F32)
    ones = jnp.ones((CHUNK, 1), F32)

    pad = EPAD - EE
    src_p = jnp.concatenate([src, jnp.zeros((pad,), jnp.int32)])
    srcR = src_p.reshape(JCP, CHUNK)
    src2R = jnp.concatenate([src_p, src_p + NN]).reshape(2 * JCP, CHUNK)
    dstR = jnp.concatenate([dst, jnp.full((pad,), NN, jnp.int32)]).reshape(JCP, CHUNK)

    f4 = _flags_deg_k()(dstR.reshape(EPAD), boundary_index, interior_index,
                        z1, ones)
    z, dinv = _stage2(x, f4,
                      W_b1, b_b1.reshape(1, 16), W_b2, b_b2.reshape(1, 16),
                      W_i1, b_i1.reshape(1, 16), W_i2, b_i2.reshape(1, 16))
    agg1 = _spmm16_k()(srcR, dstR, z, z16)
    y2 = _gcn1(agg1, z, dinv, W_c1, b_c1.reshape(1, 32), W_c2)
    acc2 = _spmm32_k()(src2R, dstR, y2.reshape(NC * NN, 16), z16)
    y3 = _mid(acc2, y2, dinv, W_c3, b_c2.reshape(1, 32))
    acc3 = _spmm1_k()(srcR, dstR, y3, z1)
    out = _final(acc3, y3, dinv, b_c3.reshape(1, 1))
    return out
